# Initial kernel scaffold; baseline (speedup 1.0000x reference)
import jax
import jax.numpy as jnp
from jax.experimental import pallas as pl


def kernel(tokens, embedding):
    raise NotImplementedError("write your pallas kernel here")


# speedup vs baseline: 1.5075x; 1.5075x over previous
import functools

import jax
import jax.numpy as jnp
from jax import lax
from jax.experimental import pallas as pl
from jax.experimental.pallas import tpu as pltpu
from jax.experimental.pallas import tpu_sc as plsc

_VOCAB = 32768
_D = 128
_B = 4 * 8192

_info = plsc.get_sparse_core_info()
_NC = _info.num_cores
_NS = _info.num_subcores
_NW = _NC * _NS

_BPW = _B // _NW
_CH = 256
_NCH = _BPW // _CH

_mesh = plsc.VectorSubcoreMesh(core_axis_name="c", subcore_axis_name="s")


@functools.partial(
    pl.kernel,
    out_type=jax.ShapeDtypeStruct((_B, _D), jnp.float32),
    mesh=_mesh,
    scratch_types=[
        pltpu.VMEM((_BPW,), jnp.int32),
        pltpu.VMEM((2, _CH, _D), jnp.float32),
        pltpu.SemaphoreType.DMA,
        pltpu.SemaphoreType.DMA,
    ],
)
def _embed_gather(tokens_hbm, table_hbm, out_hbm, idx_v, rows_v, gsem0, gsem1):
    wid = lax.axis_index("s") * _NC + lax.axis_index("c")
    base = wid * _BPW
    pltpu.sync_copy(tokens_hbm.at[pl.ds(base, _BPW)], idx_v)

    gsems = (gsem0, gsem1)
    gathers = [None, None]
    for c in range(min(2, _NCH)):
        gathers[c] = pltpu.async_copy(
            table_hbm.at[idx_v.at[pl.ds(c * _CH, _CH)]], rows_v.at[c], gsems[c]
        )
    for c in range(_NCH):
        b = c % 2
        gathers[b].wait()
        pltpu.sync_copy(rows_v.at[b], out_hbm.at[pl.ds(base + c * _CH, _CH)])
        nxt = c + 2
        if nxt < _NCH:
            gathers[b] = pltpu.async_copy(
                table_hbm.at[idx_v.at[pl.ds(nxt * _CH, _CH)]], rows_v.at[b], gsems[b]
            )


def kernel(tokens, embedding):
    flat = tokens.reshape(-1).astype(jnp.int32)
    out = _embed_gather(flat, embedding)
    return out.reshape(tokens.shape[0], tokens.shape[1], _D)


# speedup vs baseline: 1.5343x; 1.0178x over previous
import functools

import jax
import jax.numpy as jnp
from jax import lax
from jax.experimental import pallas as pl
from jax.experimental.pallas import tpu as pltpu
from jax.experimental.pallas import tpu_sc as plsc

_VOCAB = 32768
_D = 128
_B = 4 * 8192

_info = plsc.get_sparse_core_info()
_NC = _info.num_cores
_NS = _info.num_subcores
_NW = _NC * _NS

_BPW = _B // _NW
_CH = 128
_NCH = _BPW // _CH
_NB = 4
_LOOKAHEAD = 2

_mesh = plsc.VectorSubcoreMesh(core_axis_name="c", subcore_axis_name="s")


@functools.partial(
    pl.kernel,
    out_type=jax.ShapeDtypeStruct((_B, _D), jnp.float32),
    mesh=_mesh,
    scratch_types=[
        pltpu.VMEM((_BPW,), jnp.int32),
        pltpu.VMEM((_NB, _CH, _D), jnp.float32),
        [pltpu.SemaphoreType.DMA] * _NB,
        [pltpu.SemaphoreType.DMA] * _NB,
    ],
)
def _embed_gather(tokens_hbm, table_hbm, out_hbm, idx_v, rows_v, gsems, osems):
    wid = lax.axis_index("s") * _NC + lax.axis_index("c")
    base = wid * _BPW
    pltpu.sync_copy(tokens_hbm.at[pl.ds(base, _BPW)], idx_v)

    def start_gather(c, b):
        return pltpu.async_copy(
            table_hbm.at[idx_v.at[pl.ds(c * _CH, _CH)]], rows_v.at[b], gsems[b]
        )

    gathers = [None] * _NB
    outs = [None] * _NB
    for c in range(_LOOKAHEAD):
        gathers[c] = start_gather(c, c)
    for c in range(_NCH):
        b = c % _NB
        f = c + _LOOKAHEAD
        if f < _NCH:
            fb = f % _NB
            if outs[fb] is not None:
                outs[fb].wait()
            gathers[fb] = start_gather(f, fb)
        gathers[b].wait()
        outs[b] = pltpu.async_copy(
            rows_v.at[b], out_hbm.at[pl.ds(base + c * _CH, _CH)], osems[b]
        )
    for b in range(_NB):
        if outs[b] is not None:
            outs[b].wait()


def kernel(tokens, embedding):
    flat = tokens.reshape(-1).astype(jnp.int32)
    out = _embed_gather(flat, embedding)
    return out.reshape(tokens.shape[0], tokens.shape[1], _D)
